# baseline (speedup 1.0000x reference)
import functools

import jax
import jax.numpy as jnp
from jax import lax
from jax.experimental import pallas as pl
from jax.experimental.pallas import tpu as pltpu
from jax.experimental.pallas import tpu_sc as plsc

N_ROWS = 100000
N_FEAT = 12
NW = 32
ROWS_W = 3120
REM_ROWS = N_ROWS - NW * ROWS_W


def _sc_partial_sums(xp):
    mesh = plsc.VectorSubcoreMesh(core_axis_name="c", subcore_axis_name="s")

    @functools.partial(
        pl.kernel,
        mesh=mesh,
        compiler_params=pltpu.CompilerParams(
            use_tc_tiling_on_sc=False, needs_layout_passes=False
        ),
        out_type=jax.ShapeDtypeStruct((NW, 16), jnp.float32),
        scratch_types=[
            pltpu.VMEM((ROWS_W, 16), jnp.float32),
            pltpu.VMEM((16,), jnp.float32),
        ],
    )
    def k(x_hbm, out_hbm, rows_v, acc_v):
        wid = lax.axis_index("s") * 2 + lax.axis_index("c")
        base = pl.multiple_of(wid * ROWS_W, 8)
        pltpu.sync_copy(x_hbm.at[pl.ds(base, ROWS_W)], rows_v)

        def body(r, carry):
            a0, a1, a2, a3 = carry
            rr = r * 4
            return (
                a0 + rows_v[rr],
                a1 + rows_v[rr + 1],
                a2 + rows_v[rr + 2],
                a3 + rows_v[rr + 3],
            )

        zero = jnp.zeros((16,), jnp.float32)
        a0, a1, a2, a3 = lax.fori_loop(
            0, ROWS_W // 4, body, (zero, zero, zero, zero)
        )
        acc_v[...] = (a0 + a1) + (a2 + a3)
        pltpu.sync_copy(acc_v, out_hbm.at[wid])

    return k(xp)


def _tc_head(partials, x, Wa1, ba1, Wa5, ba5):
    def _kern(p_ref, xr_ref, wa1_ref, ba1_ref, wa5_ref, ba5_ref, out_ref):
        colsum = jnp.sum(p_ref[...], axis=0, keepdims=True)
        lane = lax.broadcasted_iota(jnp.int32, (16, 12), 0)
        feat = lax.broadcasted_iota(jnp.int32, (16, 12), 1)
        onehot = (lane == feat).astype(jnp.float32)
        folded = jnp.dot(colsum, onehot, preferred_element_type=jnp.float32)
        rem = jnp.sum(xr_ref[...], axis=0, keepdims=True)
        x_prime = (folded + rem) * (2.0 / N_ROWS)
        a1 = jnp.sum(wa1_ref[...] * x_prime, axis=1, keepdims=True).T
        a1 = jnp.maximum(a1 + ba1_ref[...], 0.0)
        out_ref[...] = (
            jnp.sum(a1 * wa5_ref[...], axis=1, keepdims=True) + ba5_ref[...]
        )

    return pl.pallas_call(
        _kern,
        grid=(1,),
        in_specs=[
            pl.BlockSpec((NW, 16), lambda i: (0, 0)),
            pl.BlockSpec((REM_ROWS, N_FEAT), lambda i: (NW * ROWS_W // REM_ROWS, 0)),
            pl.BlockSpec((11, 12), lambda i: (0, 0)),
            pl.BlockSpec((1, 11), lambda i: (0, 0)),
            pl.BlockSpec((1, 11), lambda i: (0, 0)),
            pl.BlockSpec((1, 1), lambda i: (0, 0)),
        ],
        out_specs=pl.BlockSpec((1, 1), lambda i: (0, 0)),
        out_shape=jax.ShapeDtypeStruct((1, 1), jnp.float32),
    )(partials, x, Wa1, ba1.reshape(1, 11), Wa5, ba5.reshape(1, 1))


def kernel(x, edge_index, W1_rel, b1_rel, W1_root, W2_rel, b2_rel, W2_root,
           Wa1, ba1, Wa5, ba5):
    del edge_index, W1_rel, b1_rel, W1_root, W2_rel, b2_rel, W2_root
    xp = jnp.pad(x, ((0, 0), (0, 4)))
    partials = _sc_partial_sums(xp)
    return _tc_head(partials, x, Wa1, ba1, Wa5, ba5)


# speedup vs baseline: 2.6083x; 2.6083x over previous
import jax
import jax.numpy as jnp
from jax.experimental import pallas as pl
from jax.experimental.pallas import tpu as pltpu

N_ROWS = 100000
N_FEAT = 12
S = 10
G = 10
BLOCK = N_ROWS // (S * G)


def _kern(*refs):
    x_refs = refs[:S]
    wa1_ref, ba1_ref, wa5_ref, ba5_ref, out_ref, acc_ref = refs[S:]
    i = pl.program_id(0)

    @pl.when(i == 0)
    def _init():
        acc_ref[...] = jnp.zeros_like(acc_ref)

    part = x_refs[0][...]
    for k in range(1, S):
        part = part + x_refs[k][...]
    acc_ref[...] += jnp.sum(part, axis=0, keepdims=True)

    @pl.when(i == pl.num_programs(0) - 1)
    def _finish():
        x_prime = acc_ref[...] * (2.0 / N_ROWS)
        a1 = jnp.sum(wa1_ref[...] * x_prime, axis=1, keepdims=True).T
        a1 = jnp.maximum(a1 + ba1_ref[...], 0.0)
        out_ref[...] = (
            jnp.sum(a1 * wa5_ref[...], axis=1, keepdims=True) + ba5_ref[...]
        )


def kernel(x, edge_index, W1_rel, b1_rel, W1_root, W2_rel, b2_rel, W2_root,
           Wa1, ba1, Wa5, ba5):
    del edge_index, W1_rel, b1_rel, W1_root, W2_rel, b2_rel, W2_root
    x_specs = [
        pl.BlockSpec((BLOCK, N_FEAT), lambda i, k=k: (i * S + k, 0))
        for k in range(S)
    ]
    return pl.pallas_call(
        _kern,
        grid=(G,),
        in_specs=x_specs + [
            pl.BlockSpec((11, 12), lambda i: (0, 0)),
            pl.BlockSpec((1, 11), lambda i: (0, 0)),
            pl.BlockSpec((1, 11), lambda i: (0, 0)),
            pl.BlockSpec((1, 1), lambda i: (0, 0)),
        ],
        out_specs=pl.BlockSpec((1, 1), lambda i: (0, 0)),
        out_shape=jax.ShapeDtypeStruct((1, 1), jnp.float32),
        scratch_shapes=[pltpu.VMEM((1, N_FEAT), jnp.float32)],
    )(*([x] * S), Wa1, ba1.reshape(1, 11), Wa5, ba5.reshape(1, 1))
